# baseline (speedup 1.0000x reference)
import functools

import jax
import jax.numpy as jnp
from jax import lax
from jax.experimental import pallas as pl
from jax.experimental.pallas import tpu as pltpu
from jax.experimental.pallas import tpu_sc as plsc

_D = 32
_BATCH = 4096
_HIST = 200
_NW = 32
_BPW = _BATCH // _NW
_CB = 8
_NCH = _BPW // _CB


def _emb_body(idx_hbm, table_hbm, out_hbm, idx_v, rows_v, sem_idx, sem_gat,
              sem_wb):
    c = lax.axis_index("c")
    s = lax.axis_index("s")
    wid = s * 2 + c
    base = wid * _BPW

    def idx_cp(i, b):
        return pltpu.make_async_copy(
            idx_hbm.at[pl.ds(base + i * _CB, _CB), :], idx_v.at[b],
            sem_idx.at[b])

    def gat_cp(b, j):
        return pltpu.make_async_copy(
            table_hbm.at[idx_v.at[b, j]], rows_v.at[b, j], sem_gat.at[b])

    def wb_cp(i, b, j):
        return pltpu.make_async_copy(
            rows_v.at[b, j], out_hbm.at[base + i * _CB + j], sem_wb.at[b])

    idx_cp(0, 0).start()
    for i in range(_NCH):
        b = i & 1
        idx_cp(i, b).wait()
        if i + 1 < _NCH:
            idx_cp(i + 1, b ^ 1).start()
        if i >= 2:
            for j in range(_CB):
                wb_cp(i - 2, b, j).wait()
        for j in range(_CB):
            gat_cp(b, j).start()
        for j in range(_CB):
            gat_cp(b, j).wait()
        for j in range(_CB):
            wb_cp(i, b, j).start()
    for i in (_NCH - 2, _NCH - 1):
        for j in range(_CB):
            wb_cp(i, i & 1, j).wait()


@functools.partial(
    pl.kernel,
    out_type=jax.ShapeDtypeStruct((_BATCH, _HIST, _D), jnp.float32),
    mesh=plsc.VectorSubcoreMesh(core_axis_name="c", subcore_axis_name="s"),
    compiler_params=pltpu.CompilerParams(use_tc_tiling_on_sc=False),
    scratch_types=[
        pltpu.VMEM((2, _CB, _HIST), jnp.int32),
        pltpu.VMEM((2, _CB, _HIST, _D), jnp.float32),
        pltpu.SemaphoreType.DMA((2,)),
        pltpu.SemaphoreType.DMA((2,)),
        pltpu.SemaphoreType.DMA((2,)),
    ],
)
def _emb(idx_hbm, table_hbm, out_hbm, idx_v, rows_v, sem_idx, sem_gat,
         sem_wb):
    _emb_body(idx_hbm, table_hbm, out_hbm, idx_v, rows_v, sem_idx, sem_gat,
              sem_wb)


def kernel(input, table):
    return _emb(input, table[1:])


# speedup vs baseline: 1.0766x; 1.0766x over previous
import functools

import jax
import jax.numpy as jnp
from jax import lax
from jax.experimental import pallas as pl
from jax.experimental.pallas import tpu as pltpu
from jax.experimental.pallas import tpu_sc as plsc

_D = 32
_BATCH = 4096
_HIST = 200
_NW = 32
_BPW = _BATCH // _NW
_CB = 8
_NCH = _BPW // _CB


def _emb_body(idx_hbm, table_hbm, out_hbm, idx_v, rows_v, sem_idx, sem_gat,
              sem_wb):
    c = lax.axis_index("c")
    s = lax.axis_index("s")
    wid = s * 2 + c
    base = wid * _BPW

    def idx_cp(i, b):
        return pltpu.make_async_copy(
            idx_hbm.at[pl.ds(base + i * _CB, _CB), :], idx_v.at[b],
            sem_idx.at[b])

    tbl = table_hbm.at[pl.ds(1, 999999)]

    def gat_cp(b, j):
        return pltpu.make_async_copy(
            tbl.at[idx_v.at[b, j]], rows_v.at[b, j], sem_gat.at[b])

    def wb_cp(i, b, j):
        return pltpu.make_async_copy(
            rows_v.at[b, j], out_hbm.at[base + i * _CB + j], sem_wb.at[b])

    idx_cp(0, 0).start()
    for i in range(_NCH):
        b = i & 1
        idx_cp(i, b).wait()
        if i + 1 < _NCH:
            idx_cp(i + 1, b ^ 1).start()
        if i >= 2:
            for j in range(_CB):
                wb_cp(i - 2, b, j).wait()
        for j in range(_CB):
            gat_cp(b, j).start()
        for j in range(_CB):
            gat_cp(b, j).wait()
        for j in range(_CB):
            wb_cp(i, b, j).start()
    for i in (_NCH - 2, _NCH - 1):
        for j in range(_CB):
            wb_cp(i, i & 1, j).wait()


@functools.partial(
    pl.kernel,
    out_type=jax.ShapeDtypeStruct((_BATCH, _HIST, _D), jnp.float32),
    mesh=plsc.VectorSubcoreMesh(core_axis_name="c", subcore_axis_name="s"),
    compiler_params=pltpu.CompilerParams(use_tc_tiling_on_sc=False),
    scratch_types=[
        pltpu.VMEM((2, _CB, _HIST), jnp.int32),
        pltpu.VMEM((2, _CB, _HIST, _D), jnp.float32),
        pltpu.SemaphoreType.DMA((2,)),
        pltpu.SemaphoreType.DMA((2,)),
        pltpu.SemaphoreType.DMA((2,)),
    ],
)
def _emb(idx_hbm, table_hbm, out_hbm, idx_v, rows_v, sem_idx, sem_gat,
         sem_wb):
    _emb_body(idx_hbm, table_hbm, out_hbm, idx_v, rows_v, sem_idx, sem_gat,
              sem_wb)


def kernel(input, table):
    return _emb(input, table)
